# Initial kernel scaffold; baseline (speedup 1.0000x reference)
import jax
import jax.numpy as jnp
from jax.experimental import pallas as pl


def kernel(voxel_score_map, depth):
    raise NotImplementedError("write your pallas kernel here")


# speedup vs baseline: 2.9712x; 2.9712x over previous
import jax
import jax.numpy as jnp
from jax import lax
from jax.experimental import pallas as pl
from jax.experimental.pallas import tpu as pltpu

_TOPK = 80
_RADIUS_GAIN = 14.0
_RADIUS_MIN = 1.5
_RADIUS_MAX = 18.0


def _shift(x, d, axis, fill):
    h, w = x.shape
    if d == 0:
        return x
    if axis == 1:
        pad = jnp.full((h, abs(d)), fill, x.dtype)
        if d > 0:
            return jnp.concatenate([pad, x[:, : w - d]], axis=1)
        return jnp.concatenate([x[:, -d:], pad], axis=1)
    pad = jnp.full((abs(d), w), fill, x.dtype)
    if d > 0:
        return jnp.concatenate([pad, x[: h - d, :]], axis=0)
    return jnp.concatenate([x[-d:, :], pad], axis=0)


def _sum9(x, axis):
    acc = x
    for d in (-4, -3, -2, -1, 1, 2, 3, 4):
        acc = acc + _shift(x, d, axis, 0.0)
    return acc


def _max5(x, axis):
    ninf = -jnp.inf
    m3 = jnp.maximum(x, jnp.maximum(_shift(x, 1, axis, ninf),
                                    _shift(x, -1, axis, ninf)))
    return jnp.maximum(_shift(m3, 1, axis, ninf), _shift(m3, -1, axis, ninf))


def _body(vs_ref, dp_ref, out_ref, mx_ref, rm_ref):
    h, w = mx_ref.shape
    p = vs_ref[0]

    occ = (p > 0.2).astype(jnp.float32)
    so = _sum9(_sum9(occ, 1), 0)
    sm = _sum9(_sum9(p, 1), 0)
    prod = so * sm * (1.0 / (81.0 * 81.0))
    mn = jnp.min(prod)
    mxv = jnp.max(prod)
    compact = (prod - mn) / (mxv - mn + 1e-6)
    score = p * compact

    pooled = _max5(_max5(score, 1), 0)
    maxima = jnp.where(score == pooled, score, 0.0)
    mx_ref[:, :] = maxima
    rm_ref[:, :] = jnp.max(maxima, axis=1, keepdims=True)

    riota = lax.broadcasted_iota(jnp.int32, (h, 1), 0)
    ciota = lax.broadcasted_iota(jnp.int32, (1, w), 1)
    yyc = riota.astype(jnp.float32)
    xxr = ciota.astype(jnp.float32)

    out_ref[0] = jnp.full((h, w), -jnp.inf, jnp.float32)

    def step(i, carry):
        rm = rm_ref[:, :]
        m = jnp.max(rm)
        r = jnp.min(jnp.where(rm == m, riota, h))
        row = mx_ref[pl.ds(r, 1), :]
        c = jnp.min(jnp.where(row == m, ciota, w))
        newrow = jnp.where(ciota == c, 0.0, row)
        mx_ref[pl.ds(r, 1), :] = newrow
        rm_ref[pl.ds(r, 1), :] = jnp.max(newrow, axis=1, keepdims=True)
        drow = dp_ref[0, pl.ds(r, 1), :]
        z = jnp.sum(jnp.where(ciota == c, drow, 0.0))
        z = jnp.maximum(z, 0.001)
        radius = jnp.clip(_RADIUS_GAIN / z, _RADIUS_MIN, _RADIUS_MAX)
        sig2 = (0.6 * radius) ** 2
        invc = 1.0 / (2.0 * sig2 + 1e-6)
        logv = jnp.log(m)
        ay = logv - (yyc - r.astype(jnp.float32)) ** 2 * invc
        bx = (xxr - c.astype(jnp.float32)) ** 2 * invc
        out_ref[0] = jnp.maximum(out_ref[0], ay - bx)
        return carry

    lax.fori_loop(0, _TOPK, step, 0, unroll=False)

    g = jnp.exp(out_ref[0])
    gmn = jnp.min(g)
    gmx = jnp.max(g)
    out_ref[0] = (g - gmn) / (gmx - gmn + 1e-6)


def kernel(voxel_score_map, depth):
    b, ch, h, w = voxel_score_map.shape
    vs = voxel_score_map.reshape(b, h, w)
    dp = depth.reshape(b, h, w)
    out = pl.pallas_call(
        _body,
        grid=(b,),
        in_specs=[
            pl.BlockSpec((1, h, w), lambda i: (i, 0, 0)),
            pl.BlockSpec((1, h, w), lambda i: (i, 0, 0)),
        ],
        out_specs=pl.BlockSpec((1, h, w), lambda i: (i, 0, 0)),
        out_shape=jax.ShapeDtypeStruct((b, h, w), jnp.float32),
        scratch_shapes=[
            pltpu.VMEM((h, w), jnp.float32),
            pltpu.VMEM((h, 1), jnp.float32),
        ],
        compiler_params=pltpu.CompilerParams(
            dimension_semantics=("arbitrary",),
        ),
    )(vs, dp)
    return out.reshape(b, ch, h, w)
